# Initial kernel scaffold; baseline (speedup 1.0000x reference)
import jax
import jax.numpy as jnp
from jax.experimental import pallas as pl


def kernel(x_cat, tables):
    raise NotImplementedError("write your pallas kernel here")


# speedup vs baseline: 1.1489x; 1.1489x over previous
import functools

import jax
import jax.numpy as jnp
from jax import lax
from jax.experimental import pallas as pl
from jax.experimental.pallas import tpu as pltpu
from jax.experimental.pallas import tpu_sc as plsc

NUM_FIELDS = 26
VOCAB = 100000
EMB_DIM = 32
BATCH = 16384

_INFO = plsc.get_sparse_core_info()
_NC, _NS, _L = _INFO.num_cores, _INFO.num_subcores, _INFO.num_lanes
_NW = _NC * _NS
_BPW = BATCH // _NW
_CHUNK = 128
_NCHUNK = _BPW // _CHUNK


def _sc_embedding(xt, flat_tables):
    mesh = plsc.VectorSubcoreMesh(core_axis_name="c", subcore_axis_name="s")

    @functools.partial(
        pl.kernel,
        mesh=mesh,
        out_type=jax.ShapeDtypeStruct((BATCH * NUM_FIELDS, EMB_DIM),
                                      jnp.float32),
        scratch_types=[
            pltpu.VMEM((NUM_FIELDS, _BPW), jnp.int32),
            pltpu.VMEM((_NCHUNK, _CHUNK), jnp.int32),
            pltpu.VMEM((_NCHUNK, _CHUNK), jnp.int32),
            pltpu.VMEM((_NCHUNK, _CHUNK), jnp.int32),
            pltpu.VMEM((_NCHUNK, _CHUNK, EMB_DIM), jnp.float32),
            pltpu.SemaphoreType.DMA,
            pltpu.SemaphoreType.DMA,
        ],
        compiler_params=pltpu.CompilerParams(use_tc_tiling_on_sc=False),
    )
    def k(xt_hbm, tab_hbm, out_hbm, idx_v, goff_v, woff_v, ramp_v, rows_v,
          gsem, wsem):
        wid = lax.axis_index("s") * _NC + lax.axis_index("c")
        base = wid * _BPW

        pltpu.sync_copy(xt_hbm.at[:, pl.ds(base, _BPW)], idx_v)

        for c in range(_NCHUNK):
            for j in range(_CHUNK // _L):
                sl = pl.ds(j * _L, _L)
                ramp_v[c, sl] = (
                    lax.iota(jnp.int32, _L) + (c * _CHUNK + j * _L)
                ) * NUM_FIELDS

        def fbody(f, carry):
            foff = f * VOCAB
            wbase = base * NUM_FIELDS + f
            for c in range(_NCHUNK):
                for j in range(_CHUNK // _L):
                    sl = pl.ds(j * _L, _L)
                    raw = idx_v[f, pl.ds(c * _CHUNK + j * _L, _L)]
                    goff_v[c, sl] = raw + foff
                    woff_v[c, sl] = ramp_v[c, sl] + wbase
            for c in range(_NCHUNK):
                pltpu.async_copy(tab_hbm.at[goff_v.at[c]], rows_v.at[c],
                                 gsem).wait()
                pltpu.async_copy(rows_v.at[c], out_hbm.at[woff_v.at[c]],
                                 wsem).wait()
            return carry

        lax.fori_loop(0, NUM_FIELDS, fbody, 0)

    return k(xt, flat_tables)


def kernel(x_cat, tables):
    xt = x_cat.T.astype(jnp.int32)
    flat = tables.reshape(NUM_FIELDS * VOCAB, EMB_DIM)
    out = _sc_embedding(xt, flat)
    return out.reshape(BATCH, NUM_FIELDS * EMB_DIM)


# speedup vs baseline: 1.2006x; 1.0450x over previous
import functools

import jax
import jax.numpy as jnp
from jax import lax
from jax.experimental import pallas as pl
from jax.experimental.pallas import tpu as pltpu
from jax.experimental.pallas import tpu_sc as plsc

NUM_FIELDS = 26
VOCAB = 100000
EMB_DIM = 32
BATCH = 16384

_INFO = plsc.get_sparse_core_info()
_NC, _NS, _L = _INFO.num_cores, _INFO.num_subcores, _INFO.num_lanes
_NW = _NC * _NS
_BPW = BATCH // _NW
_CHUNK = 512
_NCHUNK = _BPW // _CHUNK


def _sc_embedding(xt, flat_tables):
    mesh = plsc.VectorSubcoreMesh(core_axis_name="c", subcore_axis_name="s")

    @functools.partial(
        pl.kernel,
        mesh=mesh,
        out_type=jax.ShapeDtypeStruct((BATCH * NUM_FIELDS, EMB_DIM),
                                      jnp.float32),
        scratch_types=[
            pltpu.VMEM((NUM_FIELDS, _BPW), jnp.int32),
            pltpu.VMEM((_NCHUNK, _CHUNK), jnp.int32),
            pltpu.VMEM((_NCHUNK, _CHUNK), jnp.int32),
            pltpu.VMEM((_NCHUNK, _CHUNK), jnp.int32),
            pltpu.VMEM((_NCHUNK, _CHUNK, EMB_DIM), jnp.float32),
            pltpu.SemaphoreType.DMA,
            pltpu.SemaphoreType.DMA,
        ],
        compiler_params=pltpu.CompilerParams(use_tc_tiling_on_sc=False),
    )
    def k(xt_hbm, tab_hbm, out_hbm, idx_v, goff_v, woff_v, ramp_v, rows_v,
          gsem, wsem):
        wid = lax.axis_index("s") * _NC + lax.axis_index("c")
        base = wid * _BPW

        pltpu.sync_copy(xt_hbm.at[:, pl.ds(base, _BPW)], idx_v)

        for c in range(_NCHUNK):
            for j in range(_CHUNK // _L):
                sl = pl.ds(j * _L, _L)
                ramp_v[c, sl] = (
                    lax.iota(jnp.int32, _L) + (c * _CHUNK + j * _L)
                ) * NUM_FIELDS

        def fbody(f, carry):
            foff = f * VOCAB
            wbase = base * NUM_FIELDS + f
            for c in range(_NCHUNK):
                for j in range(_CHUNK // _L):
                    sl = pl.ds(j * _L, _L)
                    raw = idx_v[f, pl.ds(c * _CHUNK + j * _L, _L)]
                    goff_v[c, sl] = raw + foff
                    woff_v[c, sl] = ramp_v[c, sl] + wbase
            for c in range(_NCHUNK):
                pltpu.async_copy(tab_hbm.at[goff_v.at[c]], rows_v.at[c],
                                 gsem).wait()
                pltpu.async_copy(rows_v.at[c], out_hbm.at[woff_v.at[c]],
                                 wsem).wait()
            return carry

        lax.fori_loop(0, NUM_FIELDS, fbody, 0)

    return k(xt, flat_tables)


def kernel(x_cat, tables):
    xt = x_cat.T.astype(jnp.int32)
    flat = tables.reshape(NUM_FIELDS * VOCAB, EMB_DIM)
    out = _sc_embedding(xt, flat)
    return out.reshape(BATCH, NUM_FIELDS * EMB_DIM)
